# baseline (speedup 1.0000x reference)
import functools

import jax
import jax.numpy as jnp
from jax import lax
from jax.experimental import pallas as pl
from jax.experimental.pallas import tpu as pltpu
from jax.experimental.pallas import tpu_sc as plsc

H = 128
NBH = 6
CH = 80
NW = 32
DEPTH = 4


def _make_gather_sum(n_rows):
    n_chunks = n_rows // CH
    assert n_chunks * CH == n_rows
    n_iters = -(-n_chunks // NW)
    n_pairs = -(-n_iters // 2)
    mesh = plsc.VectorSubcoreMesh(core_axis_name="c", subcore_axis_name="s")

    @functools.partial(
        pl.kernel,
        mesh=mesh,
        out_type=jax.ShapeDtypeStruct((n_rows, H), jnp.float32),
        scratch_types=(
            [pltpu.VMEM((NBH, CH), jnp.int32) for _ in range(2)]
            + [pltpu.VMEM((CH, H), jnp.float32) for _ in range(2 * NBH)]
            + [pltpu.SemaphoreType.DMA for _ in range(4)]
        ),
    )
    def gather_sum(table_hbm, idx_hbm, out_hbm, idx_a, idx_b,
                   a0, a1, a2, a3, a4, a5, b0, b1, b2, b3, b4, b5,
                   sem_a, sem_b, wsem_a, wsem_b):
        wid = lax.axis_index("s") * 2 + lax.axis_index("c")
        sets = (
            (idx_a, (a0, a1, a2, a3, a4, a5), sem_a, wsem_a),
            (idx_b, (b0, b1, b2, b3, b4, b5), sem_b, wsem_b),
        )

        def fire(t, idx_v, bufs, sem, wsem):
            k = wid + t * NW

            @pl.when(k < n_chunks)
            def _():
                pltpu.sync_copy(idx_hbm.at[k], idx_v)
                for j in range(NBH):
                    pltpu.async_copy(table_hbm.at[idx_v.at[j]], bufs[j], sem)

        def drain(t, idx_v, bufs, sem, wsem):
            k = wid + t * NW

            @pl.when(k < n_chunks)
            def _():
                for j in range(NBH):
                    pltpu.make_async_copy(
                        table_hbm.at[idx_v.at[j]], bufs[j], sem).wait()

                def row_body(r, rcarry):
                    for c in range(H // 16):
                        sl = (r, pl.ds(c * 16, 16))
                        s = bufs[0][sl]
                        for j in range(1, NBH):
                            s = s + bufs[j][sl]
                        bufs[0][sl] = s
                    return rcarry

                lax.fori_loop(0, CH, row_body, 0)
                pltpu.async_copy(bufs[0], out_hbm.at[pl.ds(k * CH, CH)], wsem)

        def wait_write(t, idx_v, bufs, sem, wsem):
            k = wid + t * NW

            @pl.when(jnp.logical_and(k >= 0, k < n_chunks))
            def _():
                pltpu.make_async_copy(
                    bufs[0], out_hbm.at[pl.ds(k * CH, CH)], wsem).wait()

        fire(0, *sets[0])

        def pair_body(u, carry):
            t0 = 2 * u
            wait_write(t0 - 1, *sets[1])
            fire(t0 + 1, *sets[1])
            drain(t0, *sets[0])
            drain(t0 + 1, *sets[1])
            wait_write(t0, *sets[0])
            fire(t0 + 2, *sets[0])
            return carry

        lax.fori_loop(0, n_pairs, pair_body, 0)
        wait_write(2 * n_pairs - 1, *sets[1])

    return gather_sum


def _bond_input(fbonds, W_i):
    M, K = fbonds.shape
    BM = 2000

    def body(fb, wi, bi_ref, msg_ref):
        b = jnp.dot(fb[...], wi[...], preferred_element_type=jnp.float32)
        bi_ref[...] = b
        msg_ref[...] = jnp.maximum(b, 0.0)

    return pl.pallas_call(
        body,
        grid=(M // BM,),
        in_specs=[
            pl.BlockSpec((BM, K), lambda i: (i, 0)),
            pl.BlockSpec((K, H), lambda i: (0, 0)),
        ],
        out_specs=[pl.BlockSpec((BM, H), lambda i: (i, 0))] * 2,
        out_shape=[jax.ShapeDtypeStruct((M, H), jnp.float32)] * 2,
    )(fbonds, W_i)


def _update(nei, binput, W_h):
    M = nei.shape[0]
    BM = 2000

    def body(ne, bi, wh, out_ref):
        out_ref[...] = jnp.maximum(
            bi[...] + jnp.dot(ne[...], wh[...],
                              preferred_element_type=jnp.float32), 0.0)

    return pl.pallas_call(
        body,
        grid=(M // BM,),
        in_specs=[
            pl.BlockSpec((BM, H), lambda i: (i, 0)),
            pl.BlockSpec((BM, H), lambda i: (i, 0)),
            pl.BlockSpec((H, H), lambda i: (0, 0)),
        ],
        out_specs=pl.BlockSpec((BM, H), lambda i: (i, 0)),
        out_shape=jax.ShapeDtypeStruct((M, H), jnp.float32),
    )(nei, binput, W_h)


def _readout(fatoms, nei_a, W_oa, W_oh, b_o2, S):
    M, KA = fatoms.shape
    BM = 2000
    BMOL = S.shape[0]

    def body(fa, na, woa, woh, bo, s, out_ref):
        h = (jnp.dot(fa[...], woa[...], preferred_element_type=jnp.float32)
             + jnp.dot(na[...], woh[...], preferred_element_type=jnp.float32)
             + bo[...])
        h = jnp.maximum(h, 0.0)
        out_ref[...] = jnp.dot(s[...], h, preferred_element_type=jnp.float32)

    return pl.pallas_call(
        body,
        grid=(M // BM,),
        in_specs=[
            pl.BlockSpec((BM, KA), lambda i: (i, 0)),
            pl.BlockSpec((BM, H), lambda i: (i, 0)),
            pl.BlockSpec((KA, H), lambda i: (0, 0)),
            pl.BlockSpec((H, H), lambda i: (0, 0)),
            pl.BlockSpec((1, H), lambda i: (0, 0)),
            pl.BlockSpec((BMOL, BM), lambda i: (0, 0)),
        ],
        out_specs=pl.BlockSpec((BMOL, H), lambda i: (i, 0)),
        out_shape=jax.ShapeDtypeStruct((M // BM * BMOL, H), jnp.float32),
    )(fatoms, nei_a, W_oa, W_oh, b_o2, S)


def kernel(fatoms, fbonds, agraph, bgraph, ascope, bscope, W_i, W_h, W_o, b_o):
    n_atoms = fatoms.shape[0]
    n_bonds = fbonds.shape[0]
    n_mols = ascope.shape[0]
    atoms_per_mol = n_atoms // n_mols
    afdim = fatoms.shape[1]

    bidx = (bgraph.astype(jnp.int32).T
            .reshape(NBH, n_bonds // CH, CH).transpose(1, 0, 2))
    aidx = (agraph.astype(jnp.int32).T
            .reshape(NBH, n_atoms // CH, CH).transpose(1, 0, 2))

    binput, message = _bond_input(fbonds, W_i)

    gs_bonds = _make_gather_sum(n_bonds)
    for _ in range(DEPTH - 1):
        nei = gs_bonds(message, bidx)
        message = _update(nei, binput, W_h)

    gs_atoms = _make_gather_sum(n_atoms)
    nei_a = gs_atoms(message, aidx)

    W_oa = W_o[:afdim]
    W_oh = W_o[afdim:]
    b_o2 = b_o.reshape(1, H)

    BM = 2000
    mols_per_block = BM // atoms_per_mol
    S = jnp.kron(jnp.eye(mols_per_block, dtype=jnp.float32),
                 jnp.ones((1, atoms_per_mol), jnp.float32)) / atoms_per_mol

    return _readout(fatoms, nei_a, W_oa, W_oh, b_o2, S)


# speedup vs baseline: 1.1668x; 1.1668x over previous
import functools

import jax
import jax.numpy as jnp
from jax import lax
from jax.experimental import pallas as pl
from jax.experimental.pallas import tpu as pltpu
from jax.experimental.pallas import tpu_sc as plsc

H = 128
NBH = 6
CH = 40
NW = 32
DEPTH = 4


def _make_gather_sum(n_rows):
    n_chunks = n_rows // CH
    assert n_chunks * CH == n_rows
    n_iters = -(-n_chunks // NW)
    n_pairs = -(-n_iters // 2)
    mesh = plsc.VectorSubcoreMesh(core_axis_name="c", subcore_axis_name="s")

    @functools.partial(
        pl.kernel,
        mesh=mesh,
        out_type=jax.ShapeDtypeStruct((n_rows, H), jnp.float32),
        scratch_types=(
            [pltpu.VMEM((NBH, CH), jnp.int32) for _ in range(2)]
            + [pltpu.VMEM((CH, H), jnp.float32) for _ in range(2 * NBH + 2)]
            + [pltpu.SemaphoreType.DMA for _ in range(4)]
        ),
    )
    def gather_sum(table_hbm, idx_hbm, out_hbm, idx_a, idx_b,
                   a0, a1, a2, a3, a4, a5, b0, b1, b2, b3, b4, b5,
                   stage_a, stage_b, sem_a, sem_b, wsem_a, wsem_b):
        wid = lax.axis_index("s") * 2 + lax.axis_index("c")
        sets = (
            (idx_a, (a0, a1, a2, a3, a4, a5), stage_a, sem_a, wsem_a),
            (idx_b, (b0, b1, b2, b3, b4, b5), stage_b, sem_b, wsem_b),
        )

        def fire(t, idx_v, bufs, stage, sem, wsem):
            k = wid + t * NW

            @pl.when(k < n_chunks)
            def _():
                pltpu.sync_copy(idx_hbm.at[k], idx_v)
                for j in range(NBH):
                    pltpu.async_copy(table_hbm.at[idx_v.at[j]], bufs[j], sem)

        def wait_write(t, idx_v, bufs, stage, sem, wsem):
            k = wid + t * NW

            @pl.when(jnp.logical_and(k >= 0, k < n_chunks))
            def _():
                pltpu.make_async_copy(
                    stage, out_hbm.at[pl.ds(k * CH, CH)], wsem).wait()

        def drain(t, idx_v, bufs, stage, sem, wsem):
            k = wid + t * NW
            wait_write(t - 2, idx_v, bufs, stage, sem, wsem)

            @pl.when(k < n_chunks)
            def _():
                for j in range(NBH):
                    pltpu.make_async_copy(
                        table_hbm.at[idx_v.at[j]], bufs[j], sem).wait()

                def row_body(r, rcarry):
                    for c in range(H // 16):
                        sl = (r, pl.ds(c * 16, 16))
                        s = bufs[0][sl]
                        for j in range(1, NBH):
                            s = s + bufs[j][sl]
                        stage[sl] = s
                    return rcarry

                lax.fori_loop(0, CH, row_body, 0)
                pltpu.async_copy(stage, out_hbm.at[pl.ds(k * CH, CH)], wsem)

        fire(0, *sets[0])

        def pair_body(u, carry):
            t0 = 2 * u
            fire(t0 + 1, *sets[1])
            drain(t0, *sets[0])
            fire(t0 + 2, *sets[0])
            drain(t0 + 1, *sets[1])
            return carry

        lax.fori_loop(0, n_pairs, pair_body, 0)
        wait_write(2 * n_pairs - 2, *sets[0])
        wait_write(2 * n_pairs - 1, *sets[1])

    return gather_sum


def _bond_input(fbonds, W_i):
    M, K = fbonds.shape
    BM = 2000

    def body(fb, wi, bi_ref, msg_ref):
        b = jnp.dot(fb[...], wi[...], preferred_element_type=jnp.float32)
        bi_ref[...] = b
        msg_ref[...] = jnp.maximum(b, 0.0)

    return pl.pallas_call(
        body,
        grid=(M // BM,),
        in_specs=[
            pl.BlockSpec((BM, K), lambda i: (i, 0)),
            pl.BlockSpec((K, H), lambda i: (0, 0)),
        ],
        out_specs=[pl.BlockSpec((BM, H), lambda i: (i, 0))] * 2,
        out_shape=[jax.ShapeDtypeStruct((M, H), jnp.float32)] * 2,
    )(fbonds, W_i)


def _update(nei, binput, W_h):
    M = nei.shape[0]
    BM = 2000

    def body(ne, bi, wh, out_ref):
        out_ref[...] = jnp.maximum(
            bi[...] + jnp.dot(ne[...], wh[...],
                              preferred_element_type=jnp.float32), 0.0)

    return pl.pallas_call(
        body,
        grid=(M // BM,),
        in_specs=[
            pl.BlockSpec((BM, H), lambda i: (i, 0)),
            pl.BlockSpec((BM, H), lambda i: (i, 0)),
            pl.BlockSpec((H, H), lambda i: (0, 0)),
        ],
        out_specs=pl.BlockSpec((BM, H), lambda i: (i, 0)),
        out_shape=jax.ShapeDtypeStruct((M, H), jnp.float32),
    )(nei, binput, W_h)


def _readout(fatoms, nei_a, W_oa, W_oh, b_o2, S):
    M, KA = fatoms.shape
    BM = 2000
    BMOL = S.shape[0]

    def body(fa, na, woa, woh, bo, s, out_ref):
        h = (jnp.dot(fa[...], woa[...], preferred_element_type=jnp.float32)
             + jnp.dot(na[...], woh[...], preferred_element_type=jnp.float32)
             + bo[...])
        h = jnp.maximum(h, 0.0)
        out_ref[...] = jnp.dot(s[...], h, preferred_element_type=jnp.float32)

    return pl.pallas_call(
        body,
        grid=(M // BM,),
        in_specs=[
            pl.BlockSpec((BM, KA), lambda i: (i, 0)),
            pl.BlockSpec((BM, H), lambda i: (i, 0)),
            pl.BlockSpec((KA, H), lambda i: (0, 0)),
            pl.BlockSpec((H, H), lambda i: (0, 0)),
            pl.BlockSpec((1, H), lambda i: (0, 0)),
            pl.BlockSpec((BMOL, BM), lambda i: (0, 0)),
        ],
        out_specs=pl.BlockSpec((BMOL, H), lambda i: (i, 0)),
        out_shape=jax.ShapeDtypeStruct((M // BM * BMOL, H), jnp.float32),
    )(fatoms, nei_a, W_oa, W_oh, b_o2, S)


def kernel(fatoms, fbonds, agraph, bgraph, ascope, bscope, W_i, W_h, W_o, b_o):
    n_atoms = fatoms.shape[0]
    n_bonds = fbonds.shape[0]
    n_mols = ascope.shape[0]
    atoms_per_mol = n_atoms // n_mols
    afdim = fatoms.shape[1]

    bidx = (bgraph.astype(jnp.int32).T
            .reshape(NBH, n_bonds // CH, CH).transpose(1, 0, 2))
    aidx = (agraph.astype(jnp.int32).T
            .reshape(NBH, n_atoms // CH, CH).transpose(1, 0, 2))

    binput, message = _bond_input(fbonds, W_i)

    gs_bonds = _make_gather_sum(n_bonds)
    for _ in range(DEPTH - 1):
        nei = gs_bonds(message, bidx)
        message = _update(nei, binput, W_h)

    gs_atoms = _make_gather_sum(n_atoms)
    nei_a = gs_atoms(message, aidx)

    W_oa = W_o[:afdim]
    W_oh = W_o[afdim:]
    b_o2 = b_o.reshape(1, H)

    BM = 2000
    mols_per_block = BM // atoms_per_mol
    S = jnp.kron(jnp.eye(mols_per_block, dtype=jnp.float32),
                 jnp.ones((1, atoms_per_mol), jnp.float32)) / atoms_per_mol

    return _readout(fatoms, nei_a, W_oa, W_oh, b_o2, S)


# speedup vs baseline: 1.2285x; 1.0529x over previous
import functools

import jax
import jax.numpy as jnp
from jax import lax
from jax.experimental import pallas as pl
from jax.experimental.pallas import tpu as pltpu
from jax.experimental.pallas import tpu_sc as plsc

H = 128
NBH = 6
CH = 80
NW = 32
DEPTH = 4


def _make_gather_sum(n_rows):
    n_chunks = n_rows // CH
    assert n_chunks * CH == n_rows
    n_iters = -(-n_chunks // NW)
    n_pairs = -(-n_iters // 2)
    mesh = plsc.VectorSubcoreMesh(core_axis_name="c", subcore_axis_name="s")

    @functools.partial(
        pl.kernel,
        mesh=mesh,
        out_type=jax.ShapeDtypeStruct((n_rows, H), jnp.float32),
        scratch_types=(
            [pltpu.VMEM((NBH, CH), jnp.int32) for _ in range(2)]
            + [pltpu.VMEM((CH, H), jnp.float32) for _ in range(2 * NBH)]
            + [pltpu.SemaphoreType.DMA, pltpu.SemaphoreType.DMA]
        ),
    )
    def gather_sum(table_hbm, idx_hbm, out_hbm, idx_a, idx_b,
                   a0, a1, a2, a3, a4, a5, b0, b1, b2, b3, b4, b5,
                   sem_a, sem_b):
        wid = lax.axis_index("s") * 2 + lax.axis_index("c")
        sets = (
            (idx_a, (a0, a1, a2, a3, a4, a5), sem_a),
            (idx_b, (b0, b1, b2, b3, b4, b5), sem_b),
        )

        def fire(t, idx_v, bufs, sem):
            k = wid + t * NW

            @pl.when(k < n_chunks)
            def _():
                pltpu.sync_copy(idx_hbm.at[k], idx_v)
                for j in range(NBH):
                    pltpu.async_copy(table_hbm.at[idx_v.at[j]], bufs[j], sem)

        def drain(t, idx_v, bufs, sem):
            k = wid + t * NW

            @pl.when(k < n_chunks)
            def _():
                for j in range(NBH):
                    pltpu.make_async_copy(
                        table_hbm.at[idx_v.at[j]], bufs[j], sem).wait()

                def row_body(r, rcarry):
                    for c in range(H // 16):
                        sl = (r, pl.ds(c * 16, 16))
                        s = bufs[0][sl]
                        for j in range(1, NBH):
                            s = s + bufs[j][sl]
                        bufs[0][sl] = s
                    return rcarry

                lax.fori_loop(0, CH, row_body, 0)
                pltpu.sync_copy(bufs[0], out_hbm.at[pl.ds(k * CH, CH)])

        fire(0, *sets[0])

        def pair_body(u, carry):
            t0 = 2 * u
            fire(t0 + 1, *sets[1])
            drain(t0, *sets[0])
            fire(t0 + 2, *sets[0])
            drain(t0 + 1, *sets[1])
            return carry

        lax.fori_loop(0, n_pairs, pair_body, 0)

    return gather_sum


def _bond_input(fbonds, W_i):
    M, K = fbonds.shape
    BM = 2000

    def body(fb, wi, bi_ref, msg_ref):
        b = jnp.dot(fb[...], wi[...], preferred_element_type=jnp.float32)
        bi_ref[...] = b
        msg_ref[...] = jnp.maximum(b, 0.0)

    return pl.pallas_call(
        body,
        grid=(M // BM,),
        in_specs=[
            pl.BlockSpec((BM, K), lambda i: (i, 0)),
            pl.BlockSpec((K, H), lambda i: (0, 0)),
        ],
        out_specs=[pl.BlockSpec((BM, H), lambda i: (i, 0))] * 2,
        out_shape=[jax.ShapeDtypeStruct((M, H), jnp.float32)] * 2,
    )(fbonds, W_i)


def _update(nei, binput, W_h):
    M = nei.shape[0]
    BM = 2000

    def body(ne, bi, wh, out_ref):
        out_ref[...] = jnp.maximum(
            bi[...] + jnp.dot(ne[...], wh[...],
                              preferred_element_type=jnp.float32), 0.0)

    return pl.pallas_call(
        body,
        grid=(M // BM,),
        in_specs=[
            pl.BlockSpec((BM, H), lambda i: (i, 0)),
            pl.BlockSpec((BM, H), lambda i: (i, 0)),
            pl.BlockSpec((H, H), lambda i: (0, 0)),
        ],
        out_specs=pl.BlockSpec((BM, H), lambda i: (i, 0)),
        out_shape=jax.ShapeDtypeStruct((M, H), jnp.float32),
    )(nei, binput, W_h)


def _readout(fatoms, nei_a, W_oa, W_oh, b_o2, S):
    M, KA = fatoms.shape
    BM = 2000
    BMOL = S.shape[0]

    def body(fa, na, woa, woh, bo, s, out_ref):
        h = (jnp.dot(fa[...], woa[...], preferred_element_type=jnp.float32)
             + jnp.dot(na[...], woh[...], preferred_element_type=jnp.float32)
             + bo[...])
        h = jnp.maximum(h, 0.0)
        out_ref[...] = jnp.dot(s[...], h, preferred_element_type=jnp.float32)

    return pl.pallas_call(
        body,
        grid=(M // BM,),
        in_specs=[
            pl.BlockSpec((BM, KA), lambda i: (i, 0)),
            pl.BlockSpec((BM, H), lambda i: (i, 0)),
            pl.BlockSpec((KA, H), lambda i: (0, 0)),
            pl.BlockSpec((H, H), lambda i: (0, 0)),
            pl.BlockSpec((1, H), lambda i: (0, 0)),
            pl.BlockSpec((BMOL, BM), lambda i: (0, 0)),
        ],
        out_specs=pl.BlockSpec((BMOL, H), lambda i: (i, 0)),
        out_shape=jax.ShapeDtypeStruct((M // BM * BMOL, H), jnp.float32),
    )(fatoms, nei_a, W_oa, W_oh, b_o2, S)


def kernel(fatoms, fbonds, agraph, bgraph, ascope, bscope, W_i, W_h, W_o, b_o):
    n_atoms = fatoms.shape[0]
    n_bonds = fbonds.shape[0]
    n_mols = ascope.shape[0]
    atoms_per_mol = n_atoms // n_mols
    afdim = fatoms.shape[1]

    bidx = (bgraph.astype(jnp.int32).T
            .reshape(NBH, n_bonds // CH, CH).transpose(1, 0, 2))
    aidx = (agraph.astype(jnp.int32).T
            .reshape(NBH, n_atoms // CH, CH).transpose(1, 0, 2))

    binput, message = _bond_input(fbonds, W_i)

    gs_bonds = _make_gather_sum(n_bonds)
    for _ in range(DEPTH - 1):
        nei = gs_bonds(message, bidx)
        message = _update(nei, binput, W_h)

    gs_atoms = _make_gather_sum(n_atoms)
    nei_a = gs_atoms(message, aidx)

    W_oa = W_o[:afdim]
    W_oh = W_o[afdim:]
    b_o2 = b_o.reshape(1, H)

    BM = 2000
    mols_per_block = BM // atoms_per_mol
    S = jnp.kron(jnp.eye(mols_per_block, dtype=jnp.float32),
                 jnp.ones((1, atoms_per_mol), jnp.float32)) / atoms_per_mol

    return _readout(fatoms, nei_a, W_oa, W_oh, b_o2, S)


# speedup vs baseline: 1.3050x; 1.0622x over previous
import functools

import jax
import jax.numpy as jnp
from jax import lax
from jax.experimental import pallas as pl
from jax.experimental.pallas import tpu as pltpu
from jax.experimental.pallas import tpu_sc as plsc

H = 128
NBH = 6
CH = 80
NW = 32
DEPTH = 4


def _make_gather_sum(n_rows):
    n_chunks = n_rows // CH
    assert n_chunks * CH == n_rows
    n_iters = -(-n_chunks // NW)
    n_pairs = -(-n_iters // 2)
    mesh = plsc.VectorSubcoreMesh(core_axis_name="c", subcore_axis_name="s")

    @functools.partial(
        pl.kernel,
        mesh=mesh,
        out_type=jax.ShapeDtypeStruct((n_rows, H), jnp.float32),
        scratch_types=(
            [pltpu.VMEM((NBH, CH), jnp.int32) for _ in range(2)]
            + [pltpu.VMEM((CH, H), jnp.float32) for _ in range(2 * NBH)]
            + [pltpu.SemaphoreType.DMA for _ in range(4)]
        ),
    )
    def gather_sum(table_hbm, idx_hbm, out_hbm, idx_a, idx_b,
                   a0, a1, a2, a3, a4, a5, b0, b1, b2, b3, b4, b5,
                   sem_a, sem_b, isem_a, isem_b):
        wid = lax.axis_index("s") * 2 + lax.axis_index("c")
        sets = (
            (idx_a, (a0, a1, a2, a3, a4, a5), sem_a, isem_a),
            (idx_b, (b0, b1, b2, b3, b4, b5), sem_b, isem_b),
        )

        def fire(t, idx_v, bufs, sem, isem, prefetched):
            k = wid + t * NW

            @pl.when(k < n_chunks)
            def _():
                if prefetched:
                    pltpu.make_async_copy(idx_hbm.at[k], idx_v, isem).wait()
                else:
                    pltpu.sync_copy(idx_hbm.at[k], idx_v)
                for j in range(NBH):
                    pltpu.async_copy(table_hbm.at[idx_v.at[j]], bufs[j], sem)

        def drain(t, idx_v, bufs, sem, isem):
            k = wid + t * NW

            @pl.when(k < n_chunks)
            def _():
                for j in range(NBH):
                    pltpu.make_async_copy(
                        table_hbm.at[idx_v.at[j]], bufs[j], sem).wait()

                k2 = k + 2 * NW

                @pl.when(k2 < n_chunks)
                def _():
                    pltpu.async_copy(idx_hbm.at[k2], idx_v, isem)

                def row_body(r, rcarry):
                    for c in range(H // 16):
                        sl = (r, pl.ds(c * 16, 16))
                        s = bufs[0][sl]
                        for j in range(1, NBH):
                            s = s + bufs[j][sl]
                        bufs[0][sl] = s
                    return rcarry

                lax.fori_loop(0, CH, row_body, 0)
                pltpu.sync_copy(bufs[0], out_hbm.at[pl.ds(k * CH, CH)])

        def pair(t0, fire_b_pref, fire_a_pref):
            fire(t0 + 1, *sets[1], prefetched=fire_b_pref)
            drain(t0, *sets[0])
            fire(t0 + 2, *sets[0], prefetched=fire_a_pref)
            drain(t0 + 1, *sets[1])

        fire(0, *sets[0], prefetched=False)
        pair(0, fire_b_pref=False, fire_a_pref=True)

        def pair_body(u, carry):
            pair(2 * u, fire_b_pref=True, fire_a_pref=True)
            return carry

        lax.fori_loop(1, n_pairs, pair_body, 0)

    return gather_sum


def _bond_input(fbonds, W_i):
    M, K = fbonds.shape
    BM = 2000

    def body(fb, wi, bi_ref, msg_ref):
        b = jnp.dot(fb[...], wi[...], preferred_element_type=jnp.float32)
        bi_ref[...] = b
        msg_ref[...] = jnp.maximum(b, 0.0)

    return pl.pallas_call(
        body,
        grid=(M // BM,),
        in_specs=[
            pl.BlockSpec((BM, K), lambda i: (i, 0)),
            pl.BlockSpec((K, H), lambda i: (0, 0)),
        ],
        out_specs=[pl.BlockSpec((BM, H), lambda i: (i, 0))] * 2,
        out_shape=[jax.ShapeDtypeStruct((M, H), jnp.float32)] * 2,
    )(fbonds, W_i)


def _update(nei, binput, W_h):
    M = nei.shape[0]
    BM = 2000

    def body(ne, bi, wh, out_ref):
        out_ref[...] = jnp.maximum(
            bi[...] + jnp.dot(ne[...], wh[...],
                              preferred_element_type=jnp.float32), 0.0)

    return pl.pallas_call(
        body,
        grid=(M // BM,),
        in_specs=[
            pl.BlockSpec((BM, H), lambda i: (i, 0)),
            pl.BlockSpec((BM, H), lambda i: (i, 0)),
            pl.BlockSpec((H, H), lambda i: (0, 0)),
        ],
        out_specs=pl.BlockSpec((BM, H), lambda i: (i, 0)),
        out_shape=jax.ShapeDtypeStruct((M, H), jnp.float32),
    )(nei, binput, W_h)


def _readout(fatoms, nei_a, W_oa, W_oh, b_o2, S):
    M, KA = fatoms.shape
    BM = 2000
    BMOL = S.shape[0]

    def body(fa, na, woa, woh, bo, s, out_ref):
        h = (jnp.dot(fa[...], woa[...], preferred_element_type=jnp.float32)
             + jnp.dot(na[...], woh[...], preferred_element_type=jnp.float32)
             + bo[...])
        h = jnp.maximum(h, 0.0)
        out_ref[...] = jnp.dot(s[...], h, preferred_element_type=jnp.float32)

    return pl.pallas_call(
        body,
        grid=(M // BM,),
        in_specs=[
            pl.BlockSpec((BM, KA), lambda i: (i, 0)),
            pl.BlockSpec((BM, H), lambda i: (i, 0)),
            pl.BlockSpec((KA, H), lambda i: (0, 0)),
            pl.BlockSpec((H, H), lambda i: (0, 0)),
            pl.BlockSpec((1, H), lambda i: (0, 0)),
            pl.BlockSpec((BMOL, BM), lambda i: (0, 0)),
        ],
        out_specs=pl.BlockSpec((BMOL, H), lambda i: (i, 0)),
        out_shape=jax.ShapeDtypeStruct((M // BM * BMOL, H), jnp.float32),
    )(fatoms, nei_a, W_oa, W_oh, b_o2, S)


def kernel(fatoms, fbonds, agraph, bgraph, ascope, bscope, W_i, W_h, W_o, b_o):
    n_atoms = fatoms.shape[0]
    n_bonds = fbonds.shape[0]
    n_mols = ascope.shape[0]
    atoms_per_mol = n_atoms // n_mols
    afdim = fatoms.shape[1]

    bidx = (bgraph.astype(jnp.int32).T
            .reshape(NBH, n_bonds // CH, CH).transpose(1, 0, 2))
    aidx = (agraph.astype(jnp.int32).T
            .reshape(NBH, n_atoms // CH, CH).transpose(1, 0, 2))

    binput, message = _bond_input(fbonds, W_i)

    gs_bonds = _make_gather_sum(n_bonds)
    for _ in range(DEPTH - 1):
        nei = gs_bonds(message, bidx)
        message = _update(nei, binput, W_h)

    gs_atoms = _make_gather_sum(n_atoms)
    nei_a = gs_atoms(message, aidx)

    W_oa = W_o[:afdim]
    W_oh = W_o[afdim:]
    b_o2 = b_o.reshape(1, H)

    BM = 2000
    mols_per_block = BM // atoms_per_mol
    S = jnp.kron(jnp.eye(mols_per_block, dtype=jnp.float32),
                 jnp.ones((1, atoms_per_mol), jnp.float32)) / atoms_per_mol

    return _readout(fatoms, nei_a, W_oa, W_oh, b_o2, S)
